# Initial kernel scaffold; baseline (speedup 1.0000x reference)
import jax
import jax.numpy as jnp
from jax.experimental import pallas as pl


def kernel(H, edge_index, W0, W1, W2):
    raise NotImplementedError("write your pallas kernel here")


# speedup vs baseline: 1.4489x; 1.4489x over previous
import functools

import jax
import jax.numpy as jnp
from jax.experimental import pallas as pl
from jax.experimental.pallas import tpu as pltpu

K = 5


def _matmul_kernel(x_ref, w_ref, o_ref):
    o_ref[...] = jnp.dot(x_ref[...], w_ref[...],
                         preferred_element_type=jnp.float32)


def _matmul(X, Wcat, bm=1000):
    n, fin = X.shape
    fout = Wcat.shape[1]
    grid = n // bm
    return pl.pallas_call(
        _matmul_kernel,
        grid=(grid,),
        in_specs=[
            pl.BlockSpec((bm, fin), lambda i: (i, 0)),
            pl.BlockSpec((fin, fout), lambda i: (0, 0)),
        ],
        out_specs=pl.BlockSpec((bm, fout), lambda i: (i, 0)),
        out_shape=jax.ShapeDtypeStruct((n, fout), jnp.float32),
    )(X, Wcat)


def kernel(H, edge_index, W0, W1, W2):
    n, f0 = H.shape
    e_num = edge_index.shape[1]
    src = edge_index[0]
    dst = edge_index[1]

    deg = jax.ops.segment_sum(jnp.ones((e_num,), jnp.float32), dst,
                              num_segments=n)
    d = jax.lax.rsqrt(jnp.maximum(deg, 1.0))

    def spmm_raw(U):
        return jax.ops.segment_sum(U[src], dst, num_segments=n)

    dcol = d[:, None]

    def cheb_layer(X, W):
        f_out = W.shape[2]
        Wcat = jnp.concatenate([W[k] for k in range(K)], axis=1)
        Y = _matmul(X, Wcat)
        Yk = [Y[:, k * f_out:(k + 1) * f_out] for k in range(K)]
        b4 = Yk[4]
        b3 = Yk[3] - 2.0 * dcol * spmm_raw(dcol * b4)
        b2 = Yk[2] - 2.0 * dcol * spmm_raw(dcol * b3) - b4
        b1 = Yk[1] - 2.0 * dcol * spmm_raw(dcol * b2) - b3
        out = Yk[0] - dcol * spmm_raw(dcol * b1) - b2
        return out

    X = H
    feats = []
    for W in (W0, W1, W2):
        X = cheb_layer(X, W)
        feats.append(X)
        if len(feats) < 3:
            X = jax.nn.relu(X)
    logp = jax.nn.log_softmax(X, axis=1)
    return (logp, feats[0], feats[1], feats[2])


# speedup vs baseline: 10.3080x; 7.1142x over previous
import functools

import jax
import jax.numpy as jnp
from jax import lax
from jax.experimental import pallas as pl
from jax.experimental.pallas import tpu as pltpu
from jax.experimental.pallas import tpu_sc as plsc

K = 5
_NC = 2
_NS = 16
_NW = _NC * _NS
_CHUNK = 128


def _matmul_kernel(x_ref, w_ref, o_ref):
    o_ref[...] = jnp.dot(x_ref[...], w_ref[...],
                         preferred_element_type=jnp.float32)


def _matmul(X, Wcat, bm=1000):
    n, fin = X.shape
    fout = Wcat.shape[1]
    return pl.pallas_call(
        _matmul_kernel,
        grid=(n // bm,),
        in_specs=[
            pl.BlockSpec((bm, fin), lambda i: (i, 0)),
            pl.BlockSpec((fin, fout), lambda i: (0, 0)),
        ],
        out_specs=pl.BlockSpec((bm, fout), lambda i: (i, 0)),
        out_shape=jax.ShapeDtypeStruct((n, fout), jnp.float32),
    )(X, Wcat)


@functools.lru_cache(maxsize=None)
def _spmm_sc(n_acc, f, cpw):
    rps = n_acc // _NS
    mesh = plsc.VectorSubcoreMesh(core_axis_name="c", subcore_axis_name="s")

    @functools.partial(
        pl.kernel,
        out_type=jax.ShapeDtypeStruct((_NC, n_acc, f), jnp.float32),
        mesh=mesh,
        compiler_params=pltpu.CompilerParams(use_tc_tiling_on_sc=False),
        scratch_types=[
            pltpu.VMEM((cpw, _CHUNK), jnp.int32),
            pltpu.VMEM((cpw, _CHUNK), jnp.int32),
            pltpu.VMEM((_CHUNK, f), jnp.float32),
            pltpu.VMEM((rps, f), jnp.float32),
            pltpu.VMEM_SHARED((n_acc, f), jnp.float32),
            pltpu.SemaphoreType.DMA,
        ],
    )
    def spmm(u_hbm, src_hbm, dst_hbm, out_hbm,
             src_v, dst_v, rows_v, zero_v, acc_sh, gsem):
        cid = lax.axis_index("c")
        sid = lax.axis_index("s")
        wid = sid * _NC + cid
        base = wid * cpw
        pltpu.sync_copy(src_hbm.at[pl.ds(base, cpw)], src_v)
        pltpu.sync_copy(dst_hbm.at[pl.ds(base, cpw)], dst_v)

        def zrow(r, _):
            for q in range(f // 16):
                zero_v[r, pl.ds(q * 16, 16)] = jnp.zeros((16,), jnp.float32)
            return 0
        lax.fori_loop(0, rps, zrow, 0)
        pltpu.sync_copy(zero_v, acc_sh.at[pl.ds(sid * rps, rps)])
        plsc.subcore_barrier()

        def step(j, _):
            pltpu.async_copy(u_hbm.at[src_v.at[j]], rows_v, gsem).wait()
            pltpu.sync_copy(rows_v, acc_sh.at[dst_v.at[j]], add=True)
            return 0
        lax.fori_loop(0, cpw, step, 0)

        plsc.subcore_barrier()
        pltpu.sync_copy(acc_sh.at[pl.ds(sid * rps, rps)],
                        out_hbm.at[cid, pl.ds(sid * rps, rps)])

    return spmm


@functools.lru_cache(maxsize=None)
def _deg_sc(n_acc, f, cpw):
    rps = n_acc // _NS
    mesh = plsc.VectorSubcoreMesh(core_axis_name="c", subcore_axis_name="s")

    @functools.partial(
        pl.kernel,
        out_type=jax.ShapeDtypeStruct((_NC, n_acc, f), jnp.float32),
        mesh=mesh,
        compiler_params=pltpu.CompilerParams(use_tc_tiling_on_sc=False),
        scratch_types=[
            pltpu.VMEM((cpw, _CHUNK), jnp.int32),
            pltpu.VMEM((_CHUNK, f), jnp.float32),
            pltpu.VMEM((rps, f), jnp.float32),
            pltpu.VMEM_SHARED((n_acc, f), jnp.float32),
        ],
    )
    def deg(dst_hbm, out_hbm, dst_v, ones_v, zero_v, acc_sh):
        cid = lax.axis_index("c")
        sid = lax.axis_index("s")
        wid = sid * _NC + cid
        pltpu.sync_copy(dst_hbm.at[pl.ds(wid * cpw, cpw)], dst_v)

        def orow(r, _):
            for q in range(f // 16):
                ones_v[r, pl.ds(q * 16, 16)] = jnp.ones((16,), jnp.float32)
            return 0
        lax.fori_loop(0, _CHUNK, orow, 0)

        def zrow(r, _):
            for q in range(f // 16):
                zero_v[r, pl.ds(q * 16, 16)] = jnp.zeros((16,), jnp.float32)
            return 0
        lax.fori_loop(0, rps, zrow, 0)
        pltpu.sync_copy(zero_v, acc_sh.at[pl.ds(sid * rps, rps)])
        plsc.subcore_barrier()

        def step(j, _):
            pltpu.sync_copy(ones_v, acc_sh.at[dst_v.at[j]], add=True)
            return 0
        lax.fori_loop(0, cpw, step, 0)

        plsc.subcore_barrier()
        pltpu.sync_copy(acc_sh.at[pl.ds(sid * rps, rps)],
                        out_hbm.at[cid, pl.ds(sid * rps, rps)])

    return deg


def kernel(H, edge_index, W0, W1, W2):
    n, f0 = H.shape
    e_num = edge_index.shape[1]

    e_pad = ((e_num + _NW * _CHUNK - 1) // (_NW * _CHUNK)) * (_NW * _CHUNK)
    n_chunks = e_pad // _CHUNK
    cpw = n_chunks // _NW
    n_acc = ((n + 1 + _NS * 8 - 1) // (_NS * 8)) * (_NS * 8)
    pad = e_pad - e_num
    src = jnp.concatenate([edge_index[0], jnp.zeros((pad,), jnp.int32)])
    dst = jnp.concatenate([edge_index[1], jnp.full((pad,), n, jnp.int32)])
    src2d = src.reshape(n_chunks, _CHUNK)
    dst2d = dst.reshape(n_chunks, _CHUNK)

    degp = _deg_sc(n_acc, 16, cpw)(dst2d)
    deg = degp[0, :n, 0] + degp[1, :n, 0]
    d = lax.rsqrt(jnp.maximum(deg, 1.0))
    dcol = d[:, None]

    def spmm(U, f):
        p = _spmm_sc(n_acc, f, cpw)(U, src2d, dst2d)
        return p[0, :n] + p[1, :n]

    def cheb_layer(X, W):
        f = W.shape[2]
        Wcat = jnp.concatenate([W[k] for k in range(K)], axis=1)
        Y = _matmul(X, Wcat)
        Yk = [Y[:, k * f:(k + 1) * f] for k in range(K)]
        b4 = Yk[4]
        b3 = Yk[3] - 2.0 * dcol * spmm(dcol * b4, f)
        b2 = Yk[2] - 2.0 * dcol * spmm(dcol * b3, f) - b4
        b1 = Yk[1] - 2.0 * dcol * spmm(dcol * b2, f) - b3
        return Yk[0] - dcol * spmm(dcol * b1, f) - b2

    X = H
    feats = []
    for W in (W0, W1, W2):
        X = cheb_layer(X, W)
        feats.append(X)
        if len(feats) < 3:
            X = jax.nn.relu(X)
    logp = jax.nn.log_softmax(X, axis=1)
    return (logp, feats[0], feats[1], feats[2])


# speedup vs baseline: 13.4273x; 1.3026x over previous
import functools

import jax
import jax.numpy as jnp
from jax import lax
from jax.experimental import pallas as pl
from jax.experimental.pallas import tpu as pltpu
from jax.experimental.pallas import tpu_sc as plsc

K = 5
_NC = 2
_NS = 16
_NW = _NC * _NS
_CHUNK = 128


def _matmul_kernel(x_ref, w_ref, o_ref):
    o_ref[...] = jnp.dot(x_ref[...], w_ref[...],
                         preferred_element_type=jnp.float32)


def _matmul(X, Wcat, bm=1000):
    n, fin = X.shape
    fout = Wcat.shape[1]
    return pl.pallas_call(
        _matmul_kernel,
        grid=(n // bm,),
        in_specs=[
            pl.BlockSpec((bm, fin), lambda i: (i, 0)),
            pl.BlockSpec((fin, fout), lambda i: (0, 0)),
        ],
        out_specs=pl.BlockSpec((bm, fout), lambda i: (i, 0)),
        out_shape=jax.ShapeDtypeStruct((n, fout), jnp.float32),
    )(X, Wcat)


_RING = 4


@functools.lru_cache(maxsize=None)
def _spmm_sc(n_acc, f, cpw):
    rps = n_acc // _NS
    mesh = plsc.VectorSubcoreMesh(core_axis_name="c", subcore_axis_name="s")
    assert cpw % _RING == 0

    @functools.partial(
        pl.kernel,
        out_type=jax.ShapeDtypeStruct((_NC, n_acc, f), jnp.float32),
        mesh=mesh,
        compiler_params=pltpu.CompilerParams(use_tc_tiling_on_sc=False),
        scratch_types=[
            pltpu.VMEM((cpw, _CHUNK), jnp.int32),
            pltpu.VMEM((cpw, _CHUNK), jnp.int32),
            pltpu.VMEM((_RING, _CHUNK, f), jnp.float32),
            pltpu.VMEM((rps, f), jnp.float32),
            pltpu.VMEM_SHARED((n_acc, f), jnp.float32),
            pltpu.SemaphoreType.DMA,
        ] + [pltpu.SemaphoreType.DMA] * _RING,
    )
    def spmm(u_hbm, src_hbm, dst_hbm, out_hbm,
             src_v, dst_v, rows_v, zero_v, acc_sh, isem, *gsems):
        cid = lax.axis_index("c")
        sid = lax.axis_index("s")
        wid = sid * _NC + cid
        base = wid * cpw
        csrc = pltpu.async_copy(src_hbm.at[pl.ds(base, cpw)], src_v, isem)
        cdst = pltpu.async_copy(dst_hbm.at[pl.ds(base, cpw)], dst_v, isem)

        def zrow(r8, _):
            for rr in range(8):
                for q in range(f // 16):
                    zero_v[r8 * 8 + rr, pl.ds(q * 16, 16)] = (
                        jnp.zeros((16,), jnp.float32))
            return 0
        lax.fori_loop(0, rps // 8, zrow, 0)
        csrc.wait()
        cdst.wait()
        pltpu.sync_copy(zero_v, acc_sh.at[pl.ds(sid * rps, rps)])
        plsc.subcore_barrier()

        for r in range(_RING):
            pltpu.async_copy(u_hbm.at[src_v.at[r]], rows_v.at[r], gsems[r])

        def step(i, _):
            for r in range(_RING):
                j = i * _RING + r
                pltpu.make_async_copy(u_hbm.at[src_v.at[0]],
                                      rows_v.at[r], gsems[r]).wait()
                pltpu.sync_copy(rows_v.at[r], acc_sh.at[dst_v.at[j]],
                                add=True)
                nxt = j + _RING
                nxt = jnp.where(nxt >= cpw, nxt - cpw, nxt)
                pltpu.async_copy(u_hbm.at[src_v.at[nxt]], rows_v.at[r],
                                 gsems[r])
            return 0
        lax.fori_loop(0, cpw // _RING, step, 0)
        for r in range(_RING):
            pltpu.make_async_copy(u_hbm.at[src_v.at[0]],
                                  rows_v.at[r], gsems[r]).wait()

        plsc.subcore_barrier()
        pltpu.sync_copy(acc_sh.at[pl.ds(sid * rps, rps)],
                        out_hbm.at[cid, pl.ds(sid * rps, rps)])

    return spmm


@functools.lru_cache(maxsize=None)
def _deg_sc(n_acc, f, cpw):
    rps = n_acc // _NS
    mesh = plsc.VectorSubcoreMesh(core_axis_name="c", subcore_axis_name="s")

    @functools.partial(
        pl.kernel,
        out_type=jax.ShapeDtypeStruct((_NC, n_acc, f), jnp.float32),
        mesh=mesh,
        compiler_params=pltpu.CompilerParams(use_tc_tiling_on_sc=False),
        scratch_types=[
            pltpu.VMEM((cpw, _CHUNK), jnp.int32),
            pltpu.VMEM((_CHUNK, f), jnp.float32),
            pltpu.VMEM((rps, f), jnp.float32),
            pltpu.VMEM_SHARED((n_acc, f), jnp.float32),
        ],
    )
    def deg(dst_hbm, out_hbm, dst_v, ones_v, zero_v, acc_sh):
        cid = lax.axis_index("c")
        sid = lax.axis_index("s")
        wid = sid * _NC + cid
        pltpu.sync_copy(dst_hbm.at[pl.ds(wid * cpw, cpw)], dst_v)

        def orow(r8, _):
            for rr in range(8):
                for q in range(f // 16):
                    ones_v[r8 * 8 + rr, pl.ds(q * 16, 16)] = (
                        jnp.ones((16,), jnp.float32))
            return 0
        lax.fori_loop(0, _CHUNK // 8, orow, 0)

        def zrow(r8, _):
            for rr in range(8):
                for q in range(f // 16):
                    zero_v[r8 * 8 + rr, pl.ds(q * 16, 16)] = (
                        jnp.zeros((16,), jnp.float32))
            return 0
        lax.fori_loop(0, rps // 8, zrow, 0)
        pltpu.sync_copy(zero_v, acc_sh.at[pl.ds(sid * rps, rps)])
        plsc.subcore_barrier()

        def step(j, _):
            pltpu.sync_copy(ones_v, acc_sh.at[dst_v.at[j]], add=True)
            return 0
        lax.fori_loop(0, cpw, step, 0)

        plsc.subcore_barrier()
        pltpu.sync_copy(acc_sh.at[pl.ds(sid * rps, rps)],
                        out_hbm.at[cid, pl.ds(sid * rps, rps)])

    return deg


def kernel(H, edge_index, W0, W1, W2):
    n, f0 = H.shape
    e_num = edge_index.shape[1]

    e_pad = ((e_num + _NW * _CHUNK - 1) // (_NW * _CHUNK)) * (_NW * _CHUNK)
    n_chunks = e_pad // _CHUNK
    cpw = n_chunks // _NW
    n_acc = ((n + 1 + _NS * 8 - 1) // (_NS * 8)) * (_NS * 8)
    pad = e_pad - e_num
    src = jnp.concatenate([edge_index[0], jnp.zeros((pad,), jnp.int32)])
    dst = jnp.concatenate([edge_index[1], jnp.full((pad,), n, jnp.int32)])
    src2d = src.reshape(n_chunks, _CHUNK)
    dst2d = dst.reshape(n_chunks, _CHUNK)

    degp = _deg_sc(n_acc, 16, cpw)(dst2d)
    deg = degp[0, :n, 0] + degp[1, :n, 0]
    d = lax.rsqrt(jnp.maximum(deg, 1.0))
    dcol = d[:, None]

    def spmm(U, f):
        p = _spmm_sc(n_acc, f, cpw)(U, src2d, dst2d)
        return p[0, :n] + p[1, :n]

    def cheb_layer(X, W):
        f = W.shape[2]
        Wcat = jnp.concatenate([W[k] for k in range(K)], axis=1)
        Y = _matmul(X, Wcat)
        Yk = [Y[:, k * f:(k + 1) * f] for k in range(K)]
        b4 = Yk[4]
        b3 = Yk[3] - 2.0 * dcol * spmm(dcol * b4, f)
        b2 = Yk[2] - 2.0 * dcol * spmm(dcol * b3, f) - b4
        b1 = Yk[1] - 2.0 * dcol * spmm(dcol * b2, f) - b3
        return Yk[0] - dcol * spmm(dcol * b1, f) - b2

    X = H
    feats = []
    for W in (W0, W1, W2):
        X = cheb_layer(X, W)
        feats.append(X)
        if len(feats) < 3:
            X = jax.nn.relu(X)
    logp = jax.nn.log_softmax(X, axis=1)
    return (logp, feats[0], feats[1], feats[2])
